# baseline (speedup 1.0000x reference)
import functools

import jax
import jax.numpy as jnp
from jax import lax
from jax.experimental import pallas as pl
from jax.experimental.pallas import tpu as pltpu
from jax.experimental.pallas import tpu_sc as plsc

E = 64
NNZ = 131072
PDIM = 1048576
A = 8

NC = 1
NS = 16
PAIRS = NNZ // NS
TCHUNK = PAIRS // 128
SLICE = PDIM // NS
ZBUF = 4096
HALF = A // 2

GRID = 16
PBLK = PDIM // GRID


def _sc_scatter_body(lo, vals_hbm, idx_hbm, dense_hbm,
                     zero_v, val_v, idx_v, acc_sh, sem, sem2):
    s = lax.axis_index("s")

    def _z(i, carry):
        base = i * 128
        for u in range(8):
            zero_v[pl.ds(base + u * 16, 16)] = jnp.zeros((16,), jnp.float32)
        return carry
    lax.fori_loop(0, ZBUF // 128, _z, 0)

    pltpu.sync_copy(
        vals_hbm.at[pl.ds(s * TCHUNK, TCHUNK), pl.ds(lo * HALF, 2), :], val_v)
    pltpu.sync_copy(
        idx_hbm.at[pl.ds(s * TCHUNK, TCHUNK), pl.ds(lo * HALF, 2), :], idx_v)
    stg = []
    for r in range(HALF):
        def _clr(i, carry):
            pltpu.sync_copy(zero_v,
                            acc_sh.at[pl.ds(s * SLICE + i * ZBUF, ZBUF)])
            return carry
        lax.fori_loop(0, SLICE // ZBUF, _clr, 0)
        for d in stg:
            d.wait()
        stg = []
        plsc.subcore_barrier()

        def _scat(j, carry):
            pltpu.sync_copy(val_v.at[j, r % 2], acc_sh.at[idx_v.at[j, r % 2]],
                            add=True)
            return carry
        lax.fori_loop(0, TCHUNK, _scat, 0)

        if r % 2 == 1 and r < HALF - 1:
            g = (r + 1) // 2
            stg.append(pltpu.async_copy(
                vals_hbm.at[pl.ds(s * TCHUNK, TCHUNK),
                            pl.ds(lo * HALF + g * 2, 2), :],
                val_v, sem2))
            stg.append(pltpu.async_copy(
                idx_hbm.at[pl.ds(s * TCHUNK, TCHUNK),
                           pl.ds(lo * HALF + g * 2, 2), :],
                idx_v, sem2))
        plsc.subcore_barrier()

        out0 = r * PDIM + s * SLICE
        pltpu.sync_copy(acc_sh.at[pl.ds(s * SLICE, SLICE)],
                        dense_hbm.at[pl.ds(out0, SLICE)])


def _build_dense_half(vals_t, idx_t, lo):
    mesh = plsc.VectorSubcoreMesh(core_axis_name="c", subcore_axis_name="s",
                                  num_cores=NC)
    return pl.kernel(
        functools.partial(_sc_scatter_body, lo),
        out_type=jax.ShapeDtypeStruct((HALF * PDIM,), jnp.float32),
        mesh=mesh,
        scratch_types=[
            pltpu.VMEM((ZBUF,), jnp.float32),
            pltpu.VMEM((TCHUNK, 2, 128), jnp.float32),
            pltpu.VMEM((TCHUNK, 2, 128), jnp.int32),
            pltpu.VMEM_SHARED((PDIM,), jnp.float32),
            pltpu.SemaphoreType.DMA,
            pltpu.SemaphoreType.DMA,
        ],
        name=f"scatter_half{lo}",
    )(vals_t, idx_t)


_PAIRS = [(i, j) for i in range(A) for j in range(i, A)]


def _gram_loss_body(w_ref, *refs):
    row_refs = refs[:A]
    out_ref = refs[A]
    acc_ref = refs[A + 1]
    i = pl.program_id(0)

    @pl.when(i == 0)
    def _init():
        acc_ref[...] = jnp.zeros((len(_PAIRS) * 8, 128), jnp.float32)

    rows = [r[...].reshape(PBLK // 128, 128) for r in row_refs]
    for p, (a, b) in enumerate(_PAIRS):
        prod = rows[a] * rows[b]
        part = prod.reshape(PBLK // 1024, 8, 128).sum(axis=0)
        acc_ref[pl.ds(p * 8, 8), :] += part

    @pl.when(i == pl.num_programs(0) - 1)
    def _fin():
        flat = [jnp.sum(acc_ref[pl.ds(p * 8, 8), :]) for p in range(len(_PAIRS))]
        dots = {}
        for p, (a, b) in enumerate(_PAIRS):
            dots[(a, b)] = flat[p]
        norms = [jnp.sqrt(dots[(a, a)]) for a in range(A)]
        loss = jnp.float32(0.0)
        for a in range(A):
            for b in range(a + 1, A):
                cos = dots[(a, b)] / (norms[a] * norms[b] + 1e-8)
                conflict = jnp.maximum(-cos, 0.0)
                loss = loss + (w_ref[a, b] + w_ref[b, a]) * conflict
        out_ref[0, 0] = loss


def _gram_loss(w, dense0, dense1):
    def _mk_spec(e):
        return pl.BlockSpec((PBLK,), lambda i, e=e: ((e % HALF) * GRID + i,))
    return pl.pallas_call(
        _gram_loss_body,
        grid=(GRID,),
        in_specs=[pl.BlockSpec(memory_space=pltpu.SMEM)]
        + [_mk_spec(e) for e in range(A)],
        out_specs=pl.BlockSpec(memory_space=pltpu.SMEM),
        out_shape=jax.ShapeDtypeStruct((1, 1), jnp.float32),
        scratch_shapes=[pltpu.VMEM((len(_PAIRS) * 8, 128), jnp.float32)],
    )(w, *([dense0] * HALF + [dense1] * HALF))


def kernel(active_experts, alpha, grad_vals, grad_idx):
    ae = active_experts.astype(jnp.int32)
    vals_t = grad_vals[:A].reshape(8, NNZ // 128, 128).transpose(1, 0, 2)
    idx_t = grad_idx[:A].reshape(8, NNZ // 128, 128).transpose(1, 0, 2)
    dense0 = _build_dense_half(vals_t, idx_t, 0)
    dense1 = _build_dense_half(vals_t, idx_t, 1)
    strength = alpha[ae[:, None], ae[None, :]]
    w = jnp.where((strength >= 0.001) & (ae[:, None] != ae[None, :]),
                  strength, 0.0)
    loss = _gram_loss(w, dense0, dense1)
    return jnp.reshape(loss, ())


# speedup vs baseline: 1.0395x; 1.0395x over previous
import functools

import jax
import jax.numpy as jnp
from jax import lax
from jax.experimental import pallas as pl
from jax.experimental.pallas import tpu as pltpu
from jax.experimental.pallas import tpu_sc as plsc

E = 64
NNZ = 131072
PDIM = 1048576
A = 8

NC = 1
NS = 16
PAIRS = NNZ // NS
TCHUNK = PAIRS // 128
SLICE = PDIM // NS
ZBUF = 4096
HALF = A // 2

GRID = 16
PBLK = PDIM // GRID


def _sc_scatter_body(vals_hbm, idx_hbm, dense_hbm,
                     zero_v, val_v, idx_v, acc_sh, sem, sem2):
    s = lax.axis_index("s")

    def _z(i, carry):
        base = i * 128
        for u in range(8):
            zero_v[pl.ds(base + u * 16, 16)] = jnp.zeros((16,), jnp.float32)
        return carry
    lax.fori_loop(0, ZBUF // 128, _z, 0)

    pltpu.sync_copy(vals_hbm.at[pl.ds(s * TCHUNK, TCHUNK), pl.ds(0, 2), :],
                    val_v)
    pltpu.sync_copy(idx_hbm.at[pl.ds(s * TCHUNK, TCHUNK), pl.ds(0, 2), :],
                    idx_v)
    stg = []
    for r in range(A):
        def _clr(i, carry):
            pltpu.sync_copy(zero_v,
                            acc_sh.at[pl.ds(s * SLICE + i * ZBUF, ZBUF)])
            return carry
        lax.fori_loop(0, SLICE // ZBUF, _clr, 0)
        for d in stg:
            d.wait()
        stg = []
        plsc.subcore_barrier()

        def _scat(j, carry):
            pltpu.sync_copy(val_v.at[j, r % 2], acc_sh.at[idx_v.at[j, r % 2]],
                            add=True)
            return carry
        lax.fori_loop(0, TCHUNK, _scat, 0)

        if r % 2 == 1 and r < A - 1:
            g = (r + 1) // 2
            stg.append(pltpu.async_copy(
                vals_hbm.at[pl.ds(s * TCHUNK, TCHUNK), pl.ds(g * 2, 2), :],
                val_v, sem2))
            stg.append(pltpu.async_copy(
                idx_hbm.at[pl.ds(s * TCHUNK, TCHUNK), pl.ds(g * 2, 2), :],
                idx_v, sem2))
        plsc.subcore_barrier()

        out0 = r * PDIM + s * SLICE
        pltpu.sync_copy(acc_sh.at[pl.ds(s * SLICE, SLICE)],
                        dense_hbm.at[pl.ds(out0, SLICE)])


def _build_dense(vals_t, idx_t):
    mesh = plsc.VectorSubcoreMesh(core_axis_name="c", subcore_axis_name="s",
                                  num_cores=NC)
    return pl.kernel(
        _sc_scatter_body,
        out_type=jax.ShapeDtypeStruct((A * PDIM,), jnp.float32),
        mesh=mesh,
        scratch_types=[
            pltpu.VMEM((ZBUF,), jnp.float32),
            pltpu.VMEM((TCHUNK, 2, 128), jnp.float32),
            pltpu.VMEM((TCHUNK, 2, 128), jnp.int32),
            pltpu.VMEM_SHARED((PDIM,), jnp.float32),
            pltpu.SemaphoreType.DMA,
            pltpu.SemaphoreType.DMA,
        ],
    )(vals_t, idx_t)


_PAIRS = [(i, j) for i in range(A) for j in range(i, A)]


def _gram_loss_body(w_ref, *refs):
    row_refs = refs[:A]
    out_ref = refs[A]
    acc_ref = refs[A + 1]
    i = pl.program_id(0)

    @pl.when(i == 0)
    def _init():
        acc_ref[...] = jnp.zeros((len(_PAIRS) * 8, 128), jnp.float32)

    rows = [r[...].reshape(PBLK // 128, 128) for r in row_refs]
    for p, (a, b) in enumerate(_PAIRS):
        prod = rows[a] * rows[b]
        part = prod.reshape(PBLK // 1024, 8, 128).sum(axis=0)
        acc_ref[pl.ds(p * 8, 8), :] += part

    @pl.when(i == pl.num_programs(0) - 1)
    def _fin():
        flat = [jnp.sum(acc_ref[pl.ds(p * 8, 8), :]) for p in range(len(_PAIRS))]
        dots = {}
        for p, (a, b) in enumerate(_PAIRS):
            dots[(a, b)] = flat[p]
        norms = [jnp.sqrt(dots[(a, a)]) for a in range(A)]
        loss = jnp.float32(0.0)
        for a in range(A):
            for b in range(a + 1, A):
                cos = dots[(a, b)] / (norms[a] * norms[b] + 1e-8)
                conflict = jnp.maximum(-cos, 0.0)
                loss = loss + (w_ref[a, b] + w_ref[b, a]) * conflict
        out_ref[0, 0] = loss


def _gram_loss(w, dense_flat):
    def _mk_spec(e):
        return pl.BlockSpec((PBLK,), lambda i, e=e: (e * GRID + i,))
    return pl.pallas_call(
        _gram_loss_body,
        grid=(GRID,),
        in_specs=[pl.BlockSpec(memory_space=pltpu.SMEM)]
        + [_mk_spec(e) for e in range(A)],
        out_specs=pl.BlockSpec(memory_space=pltpu.SMEM),
        out_shape=jax.ShapeDtypeStruct((1, 1), jnp.float32),
        scratch_shapes=[pltpu.VMEM((len(_PAIRS) * 8, 128), jnp.float32)],
    )(w, *([dense_flat] * A))


def kernel(active_experts, alpha, grad_vals, grad_idx):
    ae = active_experts.astype(jnp.int32)
    vals_t = grad_vals[:A].reshape(8, NNZ // 128, 128).transpose(1, 0, 2)
    idx_t = grad_idx[:A].reshape(8, NNZ // 128, 128).transpose(1, 0, 2)
    dense_flat = _build_dense(vals_t, idx_t)
    strength = alpha[ae[:, None], ae[None, :]]
    w = jnp.where((strength >= 0.001) & (ae[:, None] != ae[None, :]),
                  strength, 0.0)
    loss = _gram_loss(w, dense_flat)
    return jnp.reshape(loss, ())


# speedup vs baseline: 1.0711x; 1.0304x over previous
import functools

import jax
import jax.numpy as jnp
from jax import lax
from jax.experimental import pallas as pl
from jax.experimental.pallas import tpu as pltpu
from jax.experimental.pallas import tpu_sc as plsc

E = 64
NNZ = 131072
PDIM = 1048576
A = 8

NC = 1
NS = 16
PAIRS = NNZ // NS
TCHUNK = PAIRS // 128
SLICE = PDIM // NS
ZBUF = 16384
HALF = A // 2

GRID = 16
PBLK = PDIM // GRID


def _sc_scatter_body(vals_hbm, idx_hbm, dense_hbm,
                     zero_v, val_v, idx_v, acc_sh, sem, sem2):
    s = lax.axis_index("s")

    def _z(i, carry):
        base = i * 128
        for u in range(8):
            zero_v[pl.ds(base + u * 16, 16)] = jnp.zeros((16,), jnp.float32)
        return carry
    lax.fori_loop(0, ZBUF // 128, _z, 0)

    pltpu.sync_copy(vals_hbm.at[pl.ds(s * TCHUNK, TCHUNK), pl.ds(0, 2), :],
                    val_v)
    pltpu.sync_copy(idx_hbm.at[pl.ds(s * TCHUNK, TCHUNK), pl.ds(0, 2), :],
                    idx_v)
    stg = []
    for r in range(A):
        def _clr(i, carry):
            pltpu.sync_copy(zero_v,
                            acc_sh.at[pl.ds(s * SLICE + i * ZBUF, ZBUF)])
            return carry
        lax.fori_loop(0, SLICE // ZBUF, _clr, 0)
        for d in stg:
            d.wait()
        stg = []
        plsc.subcore_barrier()

        def _scat(j, carry):
            pltpu.sync_copy(val_v.at[j, r % 2], acc_sh.at[idx_v.at[j, r % 2]],
                            add=True)
            return carry
        lax.fori_loop(0, TCHUNK, _scat, 0)

        if r % 2 == 1 and r < A - 1:
            g = (r + 1) // 2
            stg.append(pltpu.async_copy(
                vals_hbm.at[pl.ds(s * TCHUNK, TCHUNK), pl.ds(g * 2, 2), :],
                val_v, sem2))
            stg.append(pltpu.async_copy(
                idx_hbm.at[pl.ds(s * TCHUNK, TCHUNK), pl.ds(g * 2, 2), :],
                idx_v, sem2))
        plsc.subcore_barrier()

        out0 = r * PDIM + s * SLICE
        pltpu.sync_copy(acc_sh.at[pl.ds(s * SLICE, SLICE)],
                        dense_hbm.at[pl.ds(out0, SLICE)])


def _build_dense(vals_t, idx_t):
    mesh = plsc.VectorSubcoreMesh(core_axis_name="c", subcore_axis_name="s",
                                  num_cores=NC)
    return pl.kernel(
        _sc_scatter_body,
        out_type=jax.ShapeDtypeStruct((A * PDIM,), jnp.float32),
        mesh=mesh,
        scratch_types=[
            pltpu.VMEM((ZBUF,), jnp.float32),
            pltpu.VMEM((TCHUNK, 2, 128), jnp.float32),
            pltpu.VMEM((TCHUNK, 2, 128), jnp.int32),
            pltpu.VMEM_SHARED((PDIM,), jnp.float32),
            pltpu.SemaphoreType.DMA,
            pltpu.SemaphoreType.DMA,
        ],
    )(vals_t, idx_t)


_PAIRS = [(i, j) for i in range(A) for j in range(i, A)]


def _gram_loss_body(w_ref, *refs):
    row_refs = refs[:A]
    out_ref = refs[A]
    acc_ref = refs[A + 1]
    i = pl.program_id(0)

    @pl.when(i == 0)
    def _init():
        acc_ref[...] = jnp.zeros((len(_PAIRS) * 8, 128), jnp.float32)

    rows = [r[...].reshape(PBLK // 128, 128) for r in row_refs]
    for p, (a, b) in enumerate(_PAIRS):
        prod = rows[a] * rows[b]
        part = prod.reshape(PBLK // 1024, 8, 128).sum(axis=0)
        acc_ref[pl.ds(p * 8, 8), :] += part

    @pl.when(i == pl.num_programs(0) - 1)
    def _fin():
        flat = [jnp.sum(acc_ref[pl.ds(p * 8, 8), :]) for p in range(len(_PAIRS))]
        dots = {}
        for p, (a, b) in enumerate(_PAIRS):
            dots[(a, b)] = flat[p]
        norms = [jnp.sqrt(dots[(a, a)]) for a in range(A)]
        loss = jnp.float32(0.0)
        for a in range(A):
            for b in range(a + 1, A):
                cos = dots[(a, b)] / (norms[a] * norms[b] + 1e-8)
                conflict = jnp.maximum(-cos, 0.0)
                loss = loss + (w_ref[a, b] + w_ref[b, a]) * conflict
        out_ref[0, 0] = loss


def _gram_loss(w, dense_flat):
    def _mk_spec(e):
        return pl.BlockSpec((PBLK,), lambda i, e=e: (e * GRID + i,))
    return pl.pallas_call(
        _gram_loss_body,
        grid=(GRID,),
        in_specs=[pl.BlockSpec(memory_space=pltpu.SMEM)]
        + [_mk_spec(e) for e in range(A)],
        out_specs=pl.BlockSpec(memory_space=pltpu.SMEM),
        out_shape=jax.ShapeDtypeStruct((1, 1), jnp.float32),
        scratch_shapes=[pltpu.VMEM((len(_PAIRS) * 8, 128), jnp.float32)],
    )(w, *([dense_flat] * A))


def kernel(active_experts, alpha, grad_vals, grad_idx):
    ae = active_experts.astype(jnp.int32)
    vals_t = grad_vals[:A].reshape(8, NNZ // 128, 128).transpose(1, 0, 2)
    idx_t = grad_idx[:A].reshape(8, NNZ // 128, 128).transpose(1, 0, 2)
    dense_flat = _build_dense(vals_t, idx_t)
    strength = alpha[ae[:, None], ae[None, :]]
    w = jnp.where((strength >= 0.001) & (ae[:, None] != ae[None, :]),
                  strength, 0.0)
    loss = _gram_loss(w, dense_flat)
    return jnp.reshape(loss, ())


# speedup vs baseline: 1.0831x; 1.0112x over previous
import functools

import jax
import jax.numpy as jnp
from jax import lax
from jax.experimental import pallas as pl
from jax.experimental.pallas import tpu as pltpu
from jax.experimental.pallas import tpu_sc as plsc

E = 64
NNZ = 131072
PDIM = 1048576
A = 8

NC = 1
NS = 16
PAIRS = NNZ // NS
TCHUNK = PAIRS // 128
SLICE = PDIM // NS
ZBUF = 16384
HALF = A // 2

GRID = 16
PBLK = PDIM // GRID


def _sc_scatter_body(vals_hbm, idx_hbm, dense_hbm,
                     zero_v, val_v, idx_v, acc_sh, sem, sem2):
    s = lax.axis_index("s")

    def _z(i, carry):
        base = i * 128
        for u in range(8):
            zero_v[pl.ds(base + u * 16, 16)] = jnp.zeros((16,), jnp.float32)
        return carry
    lax.fori_loop(0, ZBUF // 128, _z, 0)

    pltpu.sync_copy(vals_hbm.at[pl.ds(s * TCHUNK, TCHUNK), pl.ds(0, 2), :],
                    val_v)
    pltpu.sync_copy(idx_hbm.at[pl.ds(s * TCHUNK, TCHUNK), pl.ds(0, 2), :],
                    idx_v)
    stg = []
    for r in range(A):
        clr = [pltpu.async_copy(
                   zero_v, acc_sh.at[pl.ds(s * SLICE + i * ZBUF, ZBUF)], sem)
               for i in range(SLICE // ZBUF)]
        for d in stg:
            d.wait()
        stg = []
        for d in clr:
            d.wait()
        plsc.subcore_barrier()

        def _scat(j, carry):
            pltpu.sync_copy(val_v.at[j, r % 2], acc_sh.at[idx_v.at[j, r % 2]],
                            add=True)
            return carry
        lax.fori_loop(0, TCHUNK, _scat, 0)

        if r % 2 == 1 and r < A - 1:
            g = (r + 1) // 2
            stg.append(pltpu.async_copy(
                vals_hbm.at[pl.ds(s * TCHUNK, TCHUNK), pl.ds(g * 2, 2), :],
                val_v, sem2))
            stg.append(pltpu.async_copy(
                idx_hbm.at[pl.ds(s * TCHUNK, TCHUNK), pl.ds(g * 2, 2), :],
                idx_v, sem2))
        plsc.subcore_barrier()

        out0 = r * PDIM + s * SLICE
        pltpu.sync_copy(acc_sh.at[pl.ds(s * SLICE, SLICE)],
                        dense_hbm.at[pl.ds(out0, SLICE)])


def _build_dense(vals_t, idx_t):
    mesh = plsc.VectorSubcoreMesh(core_axis_name="c", subcore_axis_name="s",
                                  num_cores=NC)
    return pl.kernel(
        _sc_scatter_body,
        out_type=jax.ShapeDtypeStruct((A * PDIM,), jnp.float32),
        mesh=mesh,
        scratch_types=[
            pltpu.VMEM((ZBUF,), jnp.float32),
            pltpu.VMEM((TCHUNK, 2, 128), jnp.float32),
            pltpu.VMEM((TCHUNK, 2, 128), jnp.int32),
            pltpu.VMEM_SHARED((PDIM,), jnp.float32),
            pltpu.SemaphoreType.DMA,
            pltpu.SemaphoreType.DMA,
        ],
    )(vals_t, idx_t)


_PAIRS = [(i, j) for i in range(A) for j in range(i, A)]


def _gram_loss_body(w_ref, *refs):
    row_refs = refs[:A]
    out_ref = refs[A]
    acc_ref = refs[A + 1]
    i = pl.program_id(0)

    @pl.when(i == 0)
    def _init():
        acc_ref[...] = jnp.zeros((len(_PAIRS) * 8, 128), jnp.float32)

    rows = [r[...].reshape(PBLK // 128, 128) for r in row_refs]
    for p, (a, b) in enumerate(_PAIRS):
        prod = rows[a] * rows[b]
        part = prod.reshape(PBLK // 1024, 8, 128).sum(axis=0)
        acc_ref[pl.ds(p * 8, 8), :] += part

    @pl.when(i == pl.num_programs(0) - 1)
    def _fin():
        flat = [jnp.sum(acc_ref[pl.ds(p * 8, 8), :]) for p in range(len(_PAIRS))]
        dots = {}
        for p, (a, b) in enumerate(_PAIRS):
            dots[(a, b)] = flat[p]
        norms = [jnp.sqrt(dots[(a, a)]) for a in range(A)]
        loss = jnp.float32(0.0)
        for a in range(A):
            for b in range(a + 1, A):
                cos = dots[(a, b)] / (norms[a] * norms[b] + 1e-8)
                conflict = jnp.maximum(-cos, 0.0)
                loss = loss + (w_ref[a, b] + w_ref[b, a]) * conflict
        out_ref[0, 0] = loss


def _gram_loss(w, dense_flat):
    def _mk_spec(e):
        return pl.BlockSpec((PBLK,), lambda i, e=e: (e * GRID + i,))
    return pl.pallas_call(
        _gram_loss_body,
        grid=(GRID,),
        in_specs=[pl.BlockSpec(memory_space=pltpu.SMEM)]
        + [_mk_spec(e) for e in range(A)],
        out_specs=pl.BlockSpec(memory_space=pltpu.SMEM),
        out_shape=jax.ShapeDtypeStruct((1, 1), jnp.float32),
        scratch_shapes=[pltpu.VMEM((len(_PAIRS) * 8, 128), jnp.float32)],
    )(w, *([dense_flat] * A))


def kernel(active_experts, alpha, grad_vals, grad_idx):
    ae = active_experts.astype(jnp.int32)
    vals_t = grad_vals[:A].reshape(8, NNZ // 128, 128).transpose(1, 0, 2)
    idx_t = grad_idx[:A].reshape(8, NNZ // 128, 128).transpose(1, 0, 2)
    dense_flat = _build_dense(vals_t, idx_t)
    strength = alpha[ae[:, None], ae[None, :]]
    w = jnp.where((strength >= 0.001) & (ae[:, None] != ae[None, :]),
                  strength, 0.0)
    loss = _gram_loss(w, dense_flat)
    return jnp.reshape(loss, ())
